# baseline (speedup 1.0000x reference)
import functools

import jax
import jax.numpy as jnp
from jax import lax
from jax.experimental import pallas as pl
from jax.experimental.pallas import tpu as pltpu
from jax.experimental.pallas import tpu_sc as plsc

B, C, T, H, W = 16, 128, 32, 28, 28
HW = H * W
KTOP = 4
NK = T - KTOP

CCHUNK = 32
NCC = C // CCHUNK

NROWS = B * C * NK
CH = 56


def _feat_body(x_ref, f_ref):
    xb = x_ref[0]
    f_ref[0] = jnp.sum(xb, axis=(2, 3)) / float(HW)


def _feat(x5):
    return pl.pallas_call(
        _feat_body,
        grid=(B, NCC),
        in_specs=[
            pl.BlockSpec((1, CCHUNK, T, H, W), lambda b, cc: (b, cc, 0, 0, 0))
        ],
        out_specs=pl.BlockSpec((1, CCHUNK, T), lambda b, cc: (b, cc, 0)),
        out_shape=jax.ShapeDtypeStruct((B, C, T), jnp.float32),
    )(x5)


def _rank_body(f_ref, w_ref, sorted_ref, idx_ref):
    fb = f_ref[...].astype(jnp.bfloat16).astype(jnp.float32)
    wb = w_ref[...].astype(jnp.bfloat16).astype(jnp.float32)
    s = jnp.zeros((B, T), jnp.float32)
    for c in range(C):
        s = s + fb[:, c, :] * wb[0, c]
    t_iota = lax.broadcasted_iota(jnp.int32, (B, T), 1)
    rank = jnp.zeros((B, T), jnp.int32)
    for tp in range(T):
        sp = s[:, tp : tp + 1]
        beats = (sp > s) | ((sp == s) & (tp < t_iota))
        rank = rank + beats.astype(jnp.int32)
    sorted_inds = jnp.zeros((B, T), jnp.int32)
    for t in range(T):
        rcol = rank[:, t : t + 1]
        sorted_inds = sorted_inds + jnp.where(rcol == t_iota, t, 0)
    sorted_ref[...] = sorted_inds
    back = sorted_inds[:, KTOP:]
    b3 = lax.broadcasted_iota(jnp.int32, (B, C, NK), 0)
    c3 = lax.broadcasted_iota(jnp.int32, (B, C, NK), 1)
    idx_ref[...] = (b3 * C + c3) * T + back[:, None, :]


def _rank(feat, w):
    return pl.pallas_call(
        _rank_body,
        out_shape=(
            jax.ShapeDtypeStruct((B, T), jnp.int32),
            jax.ShapeDtypeStruct((B, C, NK), jnp.int32),
        ),
    )(feat, w.reshape(1, C))


def _topk_body(perm_ref, x_ref, o_ref):
    del perm_ref
    j = pl.program_id(1)
    v = x_ref[0, :, 0, :, :].reshape(C, HW)
    for jj in range(KTOP):
        @pl.when(j == jj)
        def _():
            o_ref[0, :, jj, :] = v.T


def _topk(sorted_inds, x):
    return pl.pallas_call(
        _topk_body,
        grid_spec=pltpu.PrefetchScalarGridSpec(
            num_scalar_prefetch=1,
            grid=(B, KTOP),
            in_specs=[
                pl.BlockSpec(
                    (1, C, 1, H, W),
                    lambda b, j, perm: (b, 0, perm[b, j], 0, 0),
                ),
            ],
            out_specs=pl.BlockSpec(
                (1, HW, KTOP, C), lambda b, j, perm: (b, 0, 0, 0)
            ),
        ),
        out_shape=jax.ShapeDtypeStruct((B, HW, KTOP, C), jnp.float32),
    )(sorted_inds, x)


@functools.lru_cache(maxsize=None)
def _sc_parts():
    info = plsc.get_sparse_core_info()
    ncores, nsub = info.num_cores, info.num_subcores
    nw = ncores * nsub
    rows_per_w = NROWS // nw
    nchunk = rows_per_w // CH

    def body(xr_hbm, idx_hbm, out_hbm, idx_v, buf, gsem):
        wid = lax.axis_index("s") * ncores + lax.axis_index("c")
        base = wid * rows_per_w
        pltpu.sync_copy(idx_hbm.at[pl.ds(wid * nchunk, nchunk)], idx_v)

        def loop(i, carry):
            off = i * CH
            pltpu.async_copy(xr_hbm.at[idx_v.at[i]], buf, gsem).wait()
            pltpu.sync_copy(buf, out_hbm.at[pl.ds(base + off, CH)])
            return carry

        lax.fori_loop(0, nchunk, loop, 0)

    fn = pl.kernel(
        body,
        out_type=jax.ShapeDtypeStruct((NROWS, HW), jnp.float32),
        mesh=plsc.VectorSubcoreMesh(core_axis_name="c", subcore_axis_name="s"),
        scratch_types=[
            pltpu.VMEM((nchunk, CH), jnp.int32),
            pltpu.VMEM((CH, HW), jnp.float32),
            pltpu.SemaphoreType.DMA,
        ],
        compiler_params=pltpu.CompilerParams(use_tc_tiling_on_sc=False),
    )
    return fn, nw, nchunk


def kernel(x, x_cls, score_w, score_b):
    del x_cls, score_b
    x4 = x.reshape(B, C, T, HW)
    xr = x4.reshape(B * C * T, HW)
    sc_fn, nw, nchunk = _sc_parts()
    idx_back = (
        (jnp.arange(B)[:, None, None] * C + jnp.arange(C)[None, :, None]) * T
        + jnp.arange(NK)[None, None, :] + KTOP
    ).astype(jnp.int32)
    back = sc_fn(xr, idx_back.reshape(nw * nchunk, CH))
    return back.reshape(B, C, NK, H, W)
    sorted_inds, idx_back = _rank(feat, score_w)
    topk = _topk(sorted_inds, x)
    frames_topk_r = topk.reshape(B, H, W, KTOP, C)
    xr = x4.reshape(B * C * T, HW)
    sc_fn, nw, nchunk = _sc_parts()
    back = sc_fn(xr, idx_back.reshape(nw * nchunk, CH))
    frames_back = back.reshape(B, C, NK, H, W)
    return frames_topk_r, frames_back


# speedup vs baseline: 5.2942x; 5.2942x over previous
import functools

import jax
import jax.numpy as jnp
from jax import lax
from jax.experimental import pallas as pl
from jax.experimental.pallas import tpu as pltpu
from jax.experimental.pallas import tpu_sc as plsc

B, C, T, H, W = 16, 128, 32, 28, 28
HW = H * W
KTOP = 4
NK = T - KTOP

HCH = 4
NHC = H // HCH

NBROWS = NK * H * W * B
NTROWS = B * H * W * KTOP
CHR = 112


def _feat_body(x_ref, f_ref):
    hc = pl.program_id(1)

    @pl.when(hc == 0)
    def _():
        f_ref[...] = jnp.zeros_like(f_ref)

    xb = x_ref[0]
    f_ref[0] += jnp.sum(xb, axis=(0, 1)) / float(HW)


def _feat(xt):
    return pl.pallas_call(
        _feat_body,
        grid=(B, NHC),
        in_specs=[
            pl.BlockSpec((1, HCH, W, T, C), lambda b, hc: (b, hc, 0, 0, 0))
        ],
        out_specs=pl.BlockSpec((1, T, C), lambda b, hc: (b, 0, 0)),
        out_shape=jax.ShapeDtypeStruct((B, T, C), jnp.float32),
    )(xt)


def _rank_body(f_ref, w_ref, idxb_ref, idxt_ref):
    fb = f_ref[...].astype(jnp.bfloat16).astype(jnp.float32)
    wb = w_ref[...].astype(jnp.bfloat16).astype(jnp.float32)
    s = jnp.sum(fb * wb[0][None, None, :], axis=2)
    t_iota = lax.broadcasted_iota(jnp.int32, (B, T), 1)
    rank = jnp.zeros((B, T), jnp.int32)
    for tp in range(T):
        sp = s[:, tp : tp + 1]
        beats = (sp > s) | ((sp == s) & (tp < t_iota))
        rank = rank + beats.astype(jnp.int32)
    sorted_inds = jnp.zeros((B, T), jnp.int32)
    for t in range(T):
        rcol = rank[:, t : t + 1]
        sorted_inds = sorted_inds + jnp.where(rcol == t_iota, t, 0)

    st = jnp.transpose(sorted_inds[:, KTOP:], (1, 0))
    jb = lax.broadcasted_iota(jnp.int32, (NK, H, W, B), 0)
    del jb
    hb = lax.broadcasted_iota(jnp.int32, (NK, H, W, B), 1)
    wv = lax.broadcasted_iota(jnp.int32, (NK, H, W, B), 2)
    bb = lax.broadcasted_iota(jnp.int32, (NK, H, W, B), 3)
    idxb_ref[...] = ((bb * H + hb) * W + wv) * T + st[:, None, None, :]

    stt = sorted_inds[:, :KTOP]
    hb2 = lax.broadcasted_iota(jnp.int32, (B, H, W, KTOP), 1)
    wv2 = lax.broadcasted_iota(jnp.int32, (B, H, W, KTOP), 2)
    bb2 = lax.broadcasted_iota(jnp.int32, (B, H, W, KTOP), 0)
    idxt_ref[...] = ((bb2 * H + hb2) * W + wv2) * T + stt[:, None, None, :]


def _rank(feat, w):
    return pl.pallas_call(
        _rank_body,
        out_shape=(
            jax.ShapeDtypeStruct((NK, H, W, B), jnp.int32),
            jax.ShapeDtypeStruct((B, H, W, KTOP), jnp.int32),
        ),
    )(feat, w.reshape(1, C))


@functools.lru_cache(maxsize=None)
def _sc_parts():
    info = plsc.get_sparse_core_info()
    ncores, nsub = info.num_cores, info.num_subcores
    nw = ncores * nsub
    nb = NBROWS // nw
    nt = NTROWS // nw
    ncb = nb // CHR
    nct = nt // CHR

    def body(xt_hbm, idxb_hbm, idxt_hbm, outb_hbm, outt_hbm,
             idxb_v, idxt_v, buf, gsem):
        wid = lax.axis_index("s") * ncores + lax.axis_index("c")
        pltpu.sync_copy(idxb_hbm.at[pl.ds(wid * ncb, ncb)], idxb_v)
        pltpu.sync_copy(idxt_hbm.at[pl.ds(wid * nct, nct)], idxt_v)

        def loop_b(i, carry):
            pltpu.async_copy(xt_hbm.at[idxb_v.at[i]], buf, gsem).wait()
            pltpu.sync_copy(buf, outb_hbm.at[pl.ds(wid * nb + i * CHR, CHR)])
            return carry

        lax.fori_loop(0, ncb, loop_b, 0)

        def loop_t(i, carry):
            pltpu.async_copy(xt_hbm.at[idxt_v.at[i]], buf, gsem).wait()
            pltpu.sync_copy(buf, outt_hbm.at[pl.ds(wid * nt + i * CHR, CHR)])
            return carry

        lax.fori_loop(0, nct, loop_t, 0)

    fn = pl.kernel(
        body,
        out_type=(
            jax.ShapeDtypeStruct((NBROWS, C), jnp.float32),
            jax.ShapeDtypeStruct((NTROWS, C), jnp.float32),
        ),
        mesh=plsc.VectorSubcoreMesh(core_axis_name="c", subcore_axis_name="s"),
        scratch_types=[
            pltpu.VMEM((ncb, CHR), jnp.int32),
            pltpu.VMEM((nct, CHR), jnp.int32),
            pltpu.VMEM((CHR, C), jnp.float32),
            pltpu.SemaphoreType.DMA,
        ],
        compiler_params=pltpu.CompilerParams(use_tc_tiling_on_sc=False),
    )
    return fn, nw


def kernel(x, x_cls, score_w, score_b):
    del x_cls, score_b
    xt = jnp.transpose(x, (0, 3, 4, 2, 1))
    feat = _feat(xt)
    idxb4, idxt4 = _rank(feat, score_w)
    xr = xt.reshape(B * H * W * T, C)
    sc_fn, nw = _sc_parts()
    outb, outt = sc_fn(
        xr,
        idxb4.reshape(NBROWS // CHR, CHR),
        idxt4.reshape(NTROWS // CHR, CHR),
    )
    frames_topk_r = outt.reshape(B, H, W, KTOP, C)
    frames_back = outb.reshape(NK, H, W, B, C).transpose(3, 4, 0, 1, 2)
    return frames_topk_r, frames_back


# speedup vs baseline: 6.6888x; 1.2634x over previous
import functools

import jax
import jax.numpy as jnp
from jax import lax
from jax.experimental import pallas as pl
from jax.experimental.pallas import tpu as pltpu
from jax.experimental.pallas import tpu_sc as plsc

B, C, T, H, W = 16, 128, 32, 28, 28
HW = H * W
KTOP = 4
NK = T - KTOP

HCH = 4
NHC = H // HCH

NBROWS = NK * H * W * B
NTROWS = B * H * W * KTOP
CHR = 112


def _feat_body(x_ref, f_ref):
    hc = pl.program_id(1)

    @pl.when(hc == 0)
    def _():
        f_ref[...] = jnp.zeros_like(f_ref)

    xb = x_ref[0]
    f_ref[0] += jnp.sum(xb, axis=(0, 1)) / float(HW)


def _feat(xt):
    return pl.pallas_call(
        _feat_body,
        grid=(B, NHC),
        in_specs=[
            pl.BlockSpec((1, HCH, W, T, C), lambda b, hc: (b, hc, 0, 0, 0))
        ],
        out_specs=pl.BlockSpec((1, T, C), lambda b, hc: (b, 0, 0)),
        out_shape=jax.ShapeDtypeStruct((B, T, C), jnp.float32),
    )(xt)


def _rank_body(f_ref, w_ref, idxb_ref, idxt_ref):
    fb = f_ref[...].astype(jnp.bfloat16).astype(jnp.float32)
    wb = w_ref[...].astype(jnp.bfloat16).astype(jnp.float32)
    s = jnp.sum(fb * wb[0][None, None, :], axis=2)
    t_iota = lax.broadcasted_iota(jnp.int32, (B, T), 1)
    rank = jnp.zeros((B, T), jnp.int32)
    for tp in range(T):
        sp = s[:, tp : tp + 1]
        beats = (sp > s) | ((sp == s) & (tp < t_iota))
        rank = rank + beats.astype(jnp.int32)
    sorted_inds = jnp.zeros((B, T), jnp.int32)
    for t in range(T):
        rcol = rank[:, t : t + 1]
        sorted_inds = sorted_inds + jnp.where(rcol == t_iota, t, 0)

    st = jnp.transpose(sorted_inds[:, KTOP:], (1, 0))
    jb = lax.broadcasted_iota(jnp.int32, (NK, H, W, B), 0)
    del jb
    hb = lax.broadcasted_iota(jnp.int32, (NK, H, W, B), 1)
    wv = lax.broadcasted_iota(jnp.int32, (NK, H, W, B), 2)
    bb = lax.broadcasted_iota(jnp.int32, (NK, H, W, B), 3)
    idxb_ref[...] = ((bb * H + hb) * W + wv) * T + st[:, None, None, :]

    stt = sorted_inds[:, :KTOP]
    hb2 = lax.broadcasted_iota(jnp.int32, (B, H, W, KTOP), 1)
    wv2 = lax.broadcasted_iota(jnp.int32, (B, H, W, KTOP), 2)
    bb2 = lax.broadcasted_iota(jnp.int32, (B, H, W, KTOP), 0)
    idxt_ref[...] = ((bb2 * H + hb2) * W + wv2) * T + stt[:, None, None, :]


def _rank(feat, w):
    return pl.pallas_call(
        _rank_body,
        out_shape=(
            jax.ShapeDtypeStruct((NK, H, W, B), jnp.int32),
            jax.ShapeDtypeStruct((B, H, W, KTOP), jnp.int32),
        ),
    )(feat, w.reshape(1, C))


@functools.lru_cache(maxsize=None)
def _sc_parts():
    info = plsc.get_sparse_core_info()
    ncores, nsub = info.num_cores, info.num_subcores
    nw = ncores * nsub
    nb = NBROWS // nw
    nt = NTROWS // nw
    ncb = nb // CHR
    nct = nt // CHR

    def body(xt_hbm, idxb_hbm, idxt_hbm, outb_hbm, outt_hbm,
             idxb_v, idxt_v, buf0, buf1, gsem0, gsem1, wsem0, wsem1):
        wid = lax.axis_index("s") * ncores + lax.axis_index("c")
        pltpu.sync_copy(idxb_hbm.at[pl.ds(wid * ncb, ncb)], idxb_v)
        pltpu.sync_copy(idxt_hbm.at[pl.ds(wid * nct, nct)], idxt_v)

        def stream(idx_v, nchunks, out_hbm, out_base):
            def g_start(i, buf, gsem):
                pltpu.async_copy(xt_hbm.at[idx_v.at[i]], buf, gsem)

            def g_wait(buf, gsem):
                pltpu.make_async_copy(xt_hbm.at[idx_v.at[0]], buf, gsem).wait()

            def w_start(i, buf, wsem):
                pltpu.async_copy(
                    buf, out_hbm.at[pl.ds(out_base + i * CHR, CHR)], wsem
                )

            def w_wait(i, buf, wsem):
                pltpu.make_async_copy(
                    buf, out_hbm.at[pl.ds(out_base + i * CHR, CHR)], wsem
                ).wait()

            g_start(0, buf0, gsem0)

            def pairbody(p, carry):
                i = 2 * p

                @pl.when(p >= 1)
                def _():
                    w_wait(i - 1, buf1, wsem1)

                g_start(i + 1, buf1, gsem1)
                g_wait(buf0, gsem0)
                w_start(i, buf0, wsem0)

                @pl.when(i + 2 < nchunks)
                def _():
                    w_wait(i, buf0, wsem0)
                    g_start(i + 2, buf0, gsem0)

                g_wait(buf1, gsem1)
                w_start(i + 1, buf1, wsem1)
                return carry

            lax.fori_loop(0, nchunks // 2, pairbody, 0)
            w_wait(nchunks - 2, buf0, wsem0)
            w_wait(nchunks - 1, buf1, wsem1)

        stream(idxb_v, ncb, outb_hbm, wid * nb)
        stream(idxt_v, nct, outt_hbm, wid * nt)

    fn = pl.kernel(
        body,
        out_type=(
            jax.ShapeDtypeStruct((NBROWS, C), jnp.float32),
            jax.ShapeDtypeStruct((NTROWS, C), jnp.float32),
        ),
        mesh=plsc.VectorSubcoreMesh(core_axis_name="c", subcore_axis_name="s"),
        scratch_types=[
            pltpu.VMEM((ncb, CHR), jnp.int32),
            pltpu.VMEM((nct, CHR), jnp.int32),
            pltpu.VMEM((CHR, C), jnp.float32),
            pltpu.VMEM((CHR, C), jnp.float32),
            pltpu.SemaphoreType.DMA,
            pltpu.SemaphoreType.DMA,
            pltpu.SemaphoreType.DMA,
            pltpu.SemaphoreType.DMA,
        ],
        compiler_params=pltpu.CompilerParams(use_tc_tiling_on_sc=False),
    )
    return fn, nw


def kernel(x, x_cls, score_w, score_b):
    del x_cls, score_b
    xt = jnp.transpose(x, (0, 3, 4, 2, 1))
    feat = _feat(xt)
    idxb4, idxt4 = _rank(feat, score_w)
    xr = xt.reshape(B * H * W * T, C)
    sc_fn, nw = _sc_parts()
    outb, outt = sc_fn(
        xr,
        idxb4.reshape(NBROWS // CHR, CHR),
        idxt4.reshape(NTROWS // CHR, CHR),
    )
    frames_topk_r = outt.reshape(B, H, W, KTOP, C)
    frames_back = outb.reshape(NK, H, W, B, C).transpose(3, 4, 0, 1, 2)
    return frames_topk_r, frames_back


# speedup vs baseline: 7.6971x; 1.1507x over previous
import functools

import jax
import jax.numpy as jnp
from jax import lax
from jax.experimental import pallas as pl
from jax.experimental.pallas import tpu as pltpu
from jax.experimental.pallas import tpu_sc as plsc

B, C, T, H, W = 16, 128, 32, 28, 28
HW = H * W
KTOP = 4
NK = T - KTOP

HCH = 4
NHC = H // HCH

NBROWS = NK * H * W * B
NTROWS = B * H * W * KTOP
CHR = 112


def _feat_body(x_ref, f_ref):
    xb = x_ref[0]
    f_ref[0] = jnp.sum(xb, axis=(0, 1)) / float(HW)


def _feat(xt):
    return pl.pallas_call(
        _feat_body,
        grid=(B,),
        in_specs=[pl.BlockSpec((1, H, W, T, C), lambda b: (b, 0, 0, 0, 0))],
        out_specs=pl.BlockSpec((1, T, C), lambda b: (b, 0, 0)),
        out_shape=jax.ShapeDtypeStruct((B, T, C), jnp.float32),
        compiler_params=pltpu.CompilerParams(
            dimension_semantics=("arbitrary",)
        ),
    )(xt)


def _rank_body(f_ref, w_ref, idxb_ref, idxt_ref):
    fb = f_ref[...].astype(jnp.bfloat16).astype(jnp.float32)
    wb = w_ref[...].astype(jnp.bfloat16).astype(jnp.float32)
    s = jnp.sum(fb * wb[0][None, None, :], axis=2)
    t_iota = lax.broadcasted_iota(jnp.int32, (B, T), 1)
    rank = jnp.zeros((B, T), jnp.int32)
    for tp in range(T):
        sp = s[:, tp : tp + 1]
        beats = (sp > s) | ((sp == s) & (tp < t_iota))
        rank = rank + beats.astype(jnp.int32)
    sorted_inds = jnp.zeros((B, T), jnp.int32)
    for t in range(T):
        rcol = rank[:, t : t + 1]
        sorted_inds = sorted_inds + jnp.where(rcol == t_iota, t, 0)

    st = jnp.transpose(sorted_inds[:, KTOP:], (1, 0))
    jb = lax.broadcasted_iota(jnp.int32, (NK, H, W, B), 0)
    del jb
    hb = lax.broadcasted_iota(jnp.int32, (NK, H, W, B), 1)
    wv = lax.broadcasted_iota(jnp.int32, (NK, H, W, B), 2)
    bb = lax.broadcasted_iota(jnp.int32, (NK, H, W, B), 3)
    idxb_ref[...] = ((bb * H + hb) * W + wv) * T + st[:, None, None, :]

    stt = sorted_inds[:, :KTOP]
    hb2 = lax.broadcasted_iota(jnp.int32, (B, H, W, KTOP), 1)
    wv2 = lax.broadcasted_iota(jnp.int32, (B, H, W, KTOP), 2)
    bb2 = lax.broadcasted_iota(jnp.int32, (B, H, W, KTOP), 0)
    idxt_ref[...] = ((bb2 * H + hb2) * W + wv2) * T + stt[:, None, None, :]


def _rank(feat, w):
    return pl.pallas_call(
        _rank_body,
        out_shape=(
            jax.ShapeDtypeStruct((NK, H, W, B), jnp.int32),
            jax.ShapeDtypeStruct((B, H, W, KTOP), jnp.int32),
        ),
    )(feat, w.reshape(1, C))


@functools.lru_cache(maxsize=None)
def _sc_parts():
    info = plsc.get_sparse_core_info()
    ncores, nsub = info.num_cores, info.num_subcores
    nw = ncores * nsub
    nb = NBROWS // nw
    nt = NTROWS // nw
    ncb = nb // CHR
    nct = nt // CHR

    def body(xt_hbm, idxb_hbm, idxt_hbm, outb_hbm, outt_hbm,
             idxb_v, idxt_v, buf0, buf1, gsem0, gsem1, wsem0, wsem1):
        wid = lax.axis_index("s") * ncores + lax.axis_index("c")
        pltpu.sync_copy(idxb_hbm.at[pl.ds(wid * ncb, ncb)], idxb_v)
        pltpu.sync_copy(idxt_hbm.at[pl.ds(wid * nct, nct)], idxt_v)

        def stream(idx_v, nchunks, out_hbm, out_base):
            def g_start(i, buf, gsem):
                pltpu.async_copy(xt_hbm.at[idx_v.at[i]], buf, gsem)

            def g_wait(buf, gsem):
                pltpu.make_async_copy(xt_hbm.at[idx_v.at[0]], buf, gsem).wait()

            def w_start(i, buf, wsem):
                pltpu.async_copy(
                    buf, out_hbm.at[pl.ds(out_base + i * CHR, CHR)], wsem
                )

            def w_wait(i, buf, wsem):
                pltpu.make_async_copy(
                    buf, out_hbm.at[pl.ds(out_base + i * CHR, CHR)], wsem
                ).wait()

            g_start(0, buf0, gsem0)

            def pairbody(p, carry):
                i = 2 * p

                @pl.when(p >= 1)
                def _():
                    w_wait(i - 1, buf1, wsem1)

                g_start(i + 1, buf1, gsem1)
                g_wait(buf0, gsem0)
                w_start(i, buf0, wsem0)

                @pl.when(i + 2 < nchunks)
                def _():
                    w_wait(i, buf0, wsem0)
                    g_start(i + 2, buf0, gsem0)

                g_wait(buf1, gsem1)
                w_start(i + 1, buf1, wsem1)
                return carry

            lax.fori_loop(0, nchunks // 2, pairbody, 0)
            w_wait(nchunks - 2, buf0, wsem0)
            w_wait(nchunks - 1, buf1, wsem1)

        stream(idxb_v, ncb, outb_hbm, wid * nb)
        stream(idxt_v, nct, outt_hbm, wid * nt)

    fn = pl.kernel(
        body,
        out_type=(
            jax.ShapeDtypeStruct((NBROWS, C), jnp.float32),
            jax.ShapeDtypeStruct((NTROWS, C), jnp.float32),
        ),
        mesh=plsc.VectorSubcoreMesh(core_axis_name="c", subcore_axis_name="s"),
        scratch_types=[
            pltpu.VMEM((ncb, CHR), jnp.int32),
            pltpu.VMEM((nct, CHR), jnp.int32),
            pltpu.VMEM((CHR, C), jnp.float32),
            pltpu.VMEM((CHR, C), jnp.float32),
            pltpu.SemaphoreType.DMA,
            pltpu.SemaphoreType.DMA,
            pltpu.SemaphoreType.DMA,
            pltpu.SemaphoreType.DMA,
        ],
        compiler_params=pltpu.CompilerParams(use_tc_tiling_on_sc=False),
    )
    return fn, nw


def kernel(x, x_cls, score_w, score_b):
    del x_cls, score_b
    xt = jnp.transpose(x, (0, 3, 4, 2, 1))
    feat = _feat(xt)
    idxb4, idxt4 = _rank(feat, score_w)
    xr = xt.reshape(B * H * W * T, C)
    sc_fn, nw = _sc_parts()
    outb, outt = sc_fn(
        xr,
        idxb4.reshape(NBROWS // CHR, CHR),
        idxt4.reshape(NTROWS // CHR, CHR),
    )
    frames_topk_r = outt.reshape(B, H, W, KTOP, C)
    frames_back = outb.reshape(NK, H, W, B, C).transpose(3, 4, 0, 1, 2)
    return frames_topk_r, frames_back


# speedup vs baseline: 7.7010x; 1.0005x over previous
import functools

import jax
import jax.numpy as jnp
from jax import lax
from jax.experimental import pallas as pl
from jax.experimental.pallas import tpu as pltpu
from jax.experimental.pallas import tpu_sc as plsc

B, C, T, H, W = 16, 128, 32, 28, 28
HW = H * W
KTOP = 4
NK = T - KTOP

NBROWS = NK * H * W * B
NTROWS = B * H * W * KTOP
CHR = 112


def _feat_body(x_ref, f_ref):
    xb = x_ref[0]
    f_ref[0] = jnp.sum(xb, axis=(0, 1)) / float(HW)


def _feat(xt):
    return pl.pallas_call(
        _feat_body,
        grid=(B,),
        in_specs=[pl.BlockSpec((1, H, W, T, C), lambda b: (b, 0, 0, 0, 0))],
        out_specs=pl.BlockSpec((1, T, C), lambda b: (b, 0, 0)),
        out_shape=jax.ShapeDtypeStruct((B, T, C), jnp.float32),
        compiler_params=pltpu.CompilerParams(
            dimension_semantics=("arbitrary",)
        ),
    )(xt)


def _rank_body(f_ref, w_ref, idxb_ref, idxt_ref):
    fb = f_ref[...].astype(jnp.bfloat16).astype(jnp.float32)
    wb = w_ref[...].astype(jnp.bfloat16).astype(jnp.float32)
    s = jnp.sum(fb * wb[0][None, None, :], axis=2)
    t_iota = lax.broadcasted_iota(jnp.int32, (B, T), 1)
    rank = jnp.zeros((B, T), jnp.int32)
    for tp in range(T):
        sp = s[:, tp : tp + 1]
        beats = (sp > s) | ((sp == s) & (tp < t_iota))
        rank = rank + beats.astype(jnp.int32)
    sorted_inds = jnp.zeros((B, T), jnp.int32)
    for t in range(T):
        rcol = rank[:, t : t + 1]
        sorted_inds = sorted_inds + jnp.where(rcol == t_iota, t, 0)

    st = jnp.transpose(sorted_inds[:, KTOP:], (1, 0))
    hb = lax.broadcasted_iota(jnp.int32, (NK, H, W, B), 1)
    wv = lax.broadcasted_iota(jnp.int32, (NK, H, W, B), 2)
    bb = lax.broadcasted_iota(jnp.int32, (NK, H, W, B), 3)
    idxb_ref[...] = ((bb * H + hb) * W + wv) * T + st[:, None, None, :]

    stt = sorted_inds[:, :KTOP]
    hb2 = lax.broadcasted_iota(jnp.int32, (B, H, W, KTOP), 1)
    wv2 = lax.broadcasted_iota(jnp.int32, (B, H, W, KTOP), 2)
    bb2 = lax.broadcasted_iota(jnp.int32, (B, H, W, KTOP), 0)
    idxt_ref[...] = ((bb2 * H + hb2) * W + wv2) * T + stt[:, None, None, :]


def _rank(feat, w):
    return pl.pallas_call(
        _rank_body,
        out_shape=(
            jax.ShapeDtypeStruct((NK, H, W, B), jnp.int32),
            jax.ShapeDtypeStruct((B, H, W, KTOP), jnp.int32),
        ),
    )(feat, w.reshape(1, C))


@functools.lru_cache(maxsize=None)
def _sc_parts():
    info = plsc.get_sparse_core_info()
    ncores, nsub = info.num_cores, info.num_subcores
    nw = ncores * nsub
    nb = NBROWS // nw
    nt = NTROWS // nw
    ncb = nb // CHR
    nct = nt // CHR

    def body(xt_hbm, idxb_hbm, idxt_hbm, outb_hbm, outt_hbm,
             idxb_v, idxt_v, buf0, buf1, gsem0, gsem1, wsem0, wsem1):
        wid = lax.axis_index("s") * ncores + lax.axis_index("c")
        pltpu.sync_copy(idxb_hbm.at[pl.ds(wid * ncb, ncb)], idxb_v)
        pltpu.sync_copy(idxt_hbm.at[pl.ds(wid * nct, nct)], idxt_v)

        def stream(idx_v, nchunks, out_hbm, out_base):
            def g_start(i, buf, gsem):
                pltpu.async_copy(xt_hbm.at[idx_v.at[i]], buf, gsem)

            def g_wait(buf, gsem):
                pltpu.make_async_copy(xt_hbm.at[idx_v.at[0]], buf, gsem).wait()

            def w_start(i, buf, wsem):
                pltpu.async_copy(
                    buf, out_hbm.at[pl.ds(out_base + i * CHR, CHR)], wsem
                )

            def w_wait(i, buf, wsem):
                pltpu.make_async_copy(
                    buf, out_hbm.at[pl.ds(out_base + i * CHR, CHR)], wsem
                ).wait()

            g_start(0, buf0, gsem0)

            def pairbody(p, carry):
                i = 2 * p

                @pl.when(p >= 1)
                def _():
                    w_wait(i - 1, buf1, wsem1)

                g_start(i + 1, buf1, gsem1)
                g_wait(buf0, gsem0)
                w_start(i, buf0, wsem0)

                @pl.when(i + 2 < nchunks)
                def _():
                    w_wait(i, buf0, wsem0)
                    g_start(i + 2, buf0, gsem0)

                g_wait(buf1, gsem1)
                w_start(i + 1, buf1, wsem1)
                return carry

            lax.fori_loop(0, nchunks // 2, pairbody, 0)
            w_wait(nchunks - 2, buf0, wsem0)
            w_wait(nchunks - 1, buf1, wsem1)

        stream(idxb_v, ncb, outb_hbm, wid * nb)
        stream(idxt_v, nct, outt_hbm, wid * nt)

    fn = pl.kernel(
        body,
        out_type=(
            jax.ShapeDtypeStruct((NBROWS, C), jnp.float32),
            jax.ShapeDtypeStruct((NTROWS, C), jnp.float32),
        ),
        mesh=plsc.VectorSubcoreMesh(core_axis_name="c", subcore_axis_name="s"),
        scratch_types=[
            pltpu.VMEM((ncb, CHR), jnp.int32),
            pltpu.VMEM((nct, CHR), jnp.int32),
            pltpu.VMEM((CHR, C), jnp.float32),
            pltpu.VMEM((CHR, C), jnp.float32),
            pltpu.SemaphoreType.DMA,
            pltpu.SemaphoreType.DMA,
            pltpu.SemaphoreType.DMA,
            pltpu.SemaphoreType.DMA,
        ],
        compiler_params=pltpu.CompilerParams(use_tc_tiling_on_sc=False),
    )
    return fn, nw


def kernel(x, x_cls, score_w, score_b):
    del x_cls, score_b
    xt = jnp.transpose(x, (0, 3, 4, 2, 1))
    feat = _feat(xt)
    idxb4, idxt4 = _rank(feat, score_w)
    xr = xt.reshape(B * H * W * T, C)
    sc_fn, nw = _sc_parts()
    outb, outt = sc_fn(
        xr,
        idxb4.reshape(NBROWS // CHR, CHR),
        idxt4.reshape(NTROWS // CHR, CHR),
    )
    frames_topk_r = outt.reshape(B, H, W, KTOP, C)
    frames_back = outb.reshape(NK, H, W, B, C).transpose(3, 4, 0, 1, 2)
    return frames_topk_r, frames_back
